# Initial kernel scaffold; baseline (speedup 1.0000x reference)
import jax
import jax.numpy as jnp
from jax.experimental import pallas as pl


def kernel(node_data, edge_index, edge_weight, W1, b1, gamma, beta):
    raise NotImplementedError("write your pallas kernel here")


# speedup vs baseline: 13.1232x; 13.1232x over previous
import functools

import jax
import jax.numpy as jnp
from jax import lax
from jax.experimental import pallas as pl
from jax.experimental.pallas import tpu as pltpu
from jax.experimental.pallas import tpu_sc as plsc

NC = 2
NS = 16
NW = NC * NS
K = 128


def _make_agg(T, N, DH, NCH, N_pad):
    DH2 = DH // 2
    NPT = N_pad // NS
    assert NPT % K == 0 and DH2 % 16 == 0 and K % 16 == 0

    def body(node_ref, src_ref, dst_ref, w_ref, s_out, z_out,
             idx_v, dst_v, w_v, rows_v, zer_v, zst_v, s_sh, z_sh, gsem):
        cid = lax.axis_index("c")
        sid = lax.axis_index("s")
        base = sid * NPT
        zero16 = jnp.zeros((16,), jnp.float32)

        def zrow(r, c):
            for j in range(DH2 // 16):
                zer_v[r, pl.ds(j * 16, 16)] = zero16
            return c
        lax.fori_loop(0, K, zrow, 0)

        def zst(i, c):
            zst_v[pl.ds(i * 16, 16)] = zero16
            return c
        lax.fori_loop(0, NPT // 16, zst, 0)

        pltpu.sync_copy(src_ref.at[sid], idx_v)
        pltpu.sync_copy(dst_ref.at[sid], dst_v)

        cvec = jnp.full((16,), cid, jnp.int32)

        def mkidx(ci, c):
            for j in range(K // 16):
                sl = pl.ds(j * 16, 16)
                v = idx_v[ci, sl]
                idx_v[ci, sl] = v + v + cvec
            return c
        lax.fori_loop(0, NCH, mkidx, 0)

        for t in range(T):
            if t > 0:
                offN = jnp.full((16,), 2 * N, jnp.int32)

                def addoff(ci, c):
                    for j in range(K // 16):
                        sl = pl.ds(j * 16, 16)
                        idx_v[ci, sl] = idx_v[ci, sl] + offN
                    return c
                lax.fori_loop(0, NCH, addoff, 0)

            for r in range(NPT // K):
                pltpu.sync_copy(zer_v, s_sh.at[pl.ds(base + r * K, K)])

            @pl.when(cid == 0)
            def _():
                pltpu.sync_copy(zst_v, z_sh.at[pl.ds(base, NPT)])
            pltpu.sync_copy(w_ref.at[t, sid], w_v)
            plsc.subcore_barrier()

            dnums = lax.GatherDimensionNumbers(
                offset_dims=(), collapsed_slice_dims=(0,),
                start_index_map=(0,))

            def chunk(ci, c):
                pltpu.async_copy(node_ref.at[idx_v.at[ci]], rows_v, gsem).wait()

                def grp(g, c2):
                    wvec = w_v[pl.ds(ci * K + g * 16, 16)]
                    for l in range(16):
                        e = g * 16 + l
                        wl = lax.gather(
                            wvec, jnp.full((16, 1), l, jnp.int32), dnums, (1,),
                            mode=lax.GatherScatterMode.PROMISE_IN_BOUNDS)
                        for j in range(DH2 // 16):
                            sl = pl.ds(j * 16, 16)
                            rows_v[e, sl] = rows_v[e, sl] * wl
                    return c2
                lax.fori_loop(0, K // 16, grp, 0)

                pltpu.sync_copy(rows_v, s_sh.at[dst_v.at[ci]], add=True)

                @pl.when(cid == 0)
                def _():
                    pltpu.sync_copy(w_v.at[pl.ds(ci * K, K)],
                                    z_sh.at[dst_v.at[ci]], add=True)
                return c
            lax.fori_loop(0, NCH, chunk, 0)
            plsc.subcore_barrier()

            for r in range(NPT // K):
                pltpu.sync_copy(s_sh.at[pl.ds(base + r * K, K)], rows_v)
                pltpu.sync_copy(rows_v, s_out.at[t, cid, pl.ds(base + r * K, K)])

            @pl.when(cid == 0)
            def _():
                pltpu.sync_copy(z_sh.at[pl.ds(base, NPT)], zst_v)
                pltpu.sync_copy(zst_v, z_out.at[t, pl.ds(base, NPT)])
            lax.fori_loop(0, NPT // 16, zst, 0)

    mesh = plsc.VectorSubcoreMesh(core_axis_name="c", subcore_axis_name="s")
    return pl.kernel(
        body,
        out_type=[
            jax.ShapeDtypeStruct((T, NC, N_pad, DH2), jnp.float32),
            jax.ShapeDtypeStruct((T, N_pad), jnp.float32),
        ],
        mesh=mesh,
        compiler_params=pltpu.CompilerParams(use_tc_tiling_on_sc=False),
        scratch_types=[
            pltpu.VMEM((NCH, K), jnp.int32),
            pltpu.VMEM((NCH, K), jnp.int32),
            pltpu.VMEM((NCH * K,), jnp.float32),
            pltpu.VMEM((K, DH2), jnp.float32),
            pltpu.VMEM((K, DH2), jnp.float32),
            pltpu.VMEM((N_pad // NS,), jnp.float32),
            pltpu.VMEM_SHARED((N_pad, DH2), jnp.float32),
            pltpu.VMEM_SHARED((N_pad,), jnp.float32),
            pltpu.SemaphoreType.DMA,
        ],
    )


def _mm_body(x_ref, s_ref, z_ref, w1_ref, b1_ref, h_ref, st_ref, *, B, DH, N):
    DH2 = DH // 2
    t = pl.program_id(0)
    b = pl.program_id(1)
    x = x_ref[0]
    z = z_ref[0].reshape(B)
    zs = jnp.where(z == 0.0, 1.0, z)
    rz = (1.0 / zs)[:, None]
    avg0 = s_ref[0, 0] * rz
    avg1 = s_ref[0, 1] * rz
    wx = jnp.where(t == 0, w1_ref[0], w1_ref[0] + w1_ref[1])
    h = (jnp.dot(x, wx, preferred_element_type=jnp.float32)
         + jnp.dot(avg0, w1_ref[2, :DH2, :], preferred_element_type=jnp.float32)
         + jnp.dot(avg1, w1_ref[2, DH2:, :], preferred_element_type=jnp.float32)
         + b1_ref[0])
    h_ref[0] = h

    @pl.when((t == 0) & (b == 0))
    def _():
        st_ref[...] = jnp.zeros_like(st_ref)
    rows = lax.broadcasted_iota(jnp.int32, (B, 1), 0)
    hm = jnp.where(rows < N - b * B, h, 0.0)
    st_ref[0, :] += jnp.sum(hm, axis=0)
    st_ref[1, :] += jnp.sum(hm * hm, axis=0)


def _bn_body(h_ref, st_ref, g_ref, be_ref, o_ref, *, M):
    h = h_ref[0]
    mean = st_ref[0, :] * (1.0 / M)
    var = st_ref[1, :] * (1.0 / M) - mean * mean
    inv = lax.rsqrt(var + 1e-5) * g_ref[0]
    o_ref[0] = jnp.maximum((h - mean) * inv + be_ref[0], 0.0)


def kernel(node_data, edge_index, edge_weight, W1, b1, gamma, beta):
    T, N, DH = node_data.shape
    E = edge_index.shape[1]
    NCH = -(-E // (NS * K))
    E_pad = NS * K * NCH
    N_pad = -(-N // (NS * K)) * (NS * K)

    dst = edge_index[0]
    src = edge_index[1]
    pad = E_pad - E
    if pad:
        src = jnp.concatenate([src, jnp.zeros((pad,), jnp.int32)])
        dst = jnp.concatenate([dst, jnp.zeros((pad,), jnp.int32)])
        edge_weight = jnp.concatenate(
            [edge_weight, jnp.zeros((T, pad), jnp.float32)], axis=1)
    src = src.reshape(NS, NCH, K)
    dst = dst.reshape(NS, NCH, K)
    w = edge_weight.reshape(T, NS, NCH * K)
    node_half = node_data.reshape(T * N * 2, DH // 2)

    s_part, z_part = _make_agg(T, N, DH, NCH, N_pad)(node_half, src, dst, w)

    B = 2048
    NB = N_pad // B
    assert N_pad % B == 0 and B % 128 == 0
    z3 = z_part.reshape(T, N_pad // 128, 128)
    w1s = W1.reshape(3, DH, DH)
    h, st = pl.pallas_call(
        functools.partial(_mm_body, B=B, DH=DH, N=N),
        grid=(T, NB),
        in_specs=[
            pl.BlockSpec((1, B, DH), lambda t, b: (t, b, 0)),
            pl.BlockSpec((1, NC, B, DH // 2), lambda t, b: (t, 0, b, 0)),
            pl.BlockSpec((1, B // 128, 128), lambda t, b: (t, b, 0)),
            pl.BlockSpec((3, DH, DH), lambda t, b: (0, 0, 0)),
            pl.BlockSpec((1, DH), lambda t, b: (0, 0)),
        ],
        out_specs=[
            pl.BlockSpec((1, B, DH), lambda t, b: (t, b, 0)),
            pl.BlockSpec((2, DH), lambda t, b: (0, 0)),
        ],
        out_shape=[
            jax.ShapeDtypeStruct((T, N, DH), jnp.float32),
            jax.ShapeDtypeStruct((2, DH), jnp.float32),
        ],
    )(node_data, s_part, z3, w1s, b1.reshape(1, DH))

    out = pl.pallas_call(
        functools.partial(_bn_body, M=float(T * N)),
        grid=(T, NB),
        in_specs=[
            pl.BlockSpec((1, B, DH), lambda t, b: (t, b, 0)),
            pl.BlockSpec((2, DH), lambda t, b: (0, 0)),
            pl.BlockSpec((1, DH), lambda t, b: (0, 0)),
            pl.BlockSpec((1, DH), lambda t, b: (0, 0)),
        ],
        out_specs=pl.BlockSpec((1, B, DH), lambda t, b: (t, b, 0)),
        out_shape=jax.ShapeDtypeStruct((T, N, DH), jnp.float32),
    )(h, st, gamma.reshape(1, DH), beta.reshape(1, DH))
    return out
